# baseline (device time: 33281 ns/iter reference)
import jax
import jax.numpy as jnp
from jax import lax
from jax.experimental import pallas as pl
from jax.experimental.pallas import tpu as pltpu

Z = 4
ECAP = 80
PCAP = 2 * ECAP


def kernel(x, assign, W1, W2):
    T, D = x.shape
    E, _, F = W1.shape
    NE = Z * E

    eq = assign[:, None] == assign[None, :]
    before = jnp.arange(T)[:, None] > jnp.arange(T)[None, :]
    rank = jnp.sum(eq & before, axis=1)
    p8 = (
        (assign[None, None, :] == jnp.arange(NE)[:, None, None])
        & (rank[None, None, :] == jnp.arange(ECAP)[None, :, None])
    ).astype(jnp.bfloat16)
    ppair = p8.reshape(Z, PCAP, T)

    def body(x_ref, w1_ref, w2_ref, pp_ref, out_ref,
             sbuf, abuf, obuf, rbuf, w1v, w2v,
             fwd_send, fwd_recv, ret_send, ret_recv, wsem):
        mx = lax.axis_index("x")
        my = lax.axis_index("y")
        mz = lax.axis_index("z")

        w1_dma = pltpu.make_async_copy(w1_ref, w1v, wsem.at[0])
        w2_dma = pltpu.make_async_copy(w2_ref, w2v, wsem.at[1])
        w1_dma.start()
        w2_dma.start()

        barrier = pltpu.get_barrier_semaphore()
        for d in range(1, Z):
            peer = lax.rem(mz + d, Z)
            pl.semaphore_signal(
                barrier, inc=1,
                device_id=(mx, my, peer),
                device_id_type=pl.DeviceIdType.MESH,
            )
        pl.semaphore_wait(barrier, Z - 1)

        xl = x_ref[...].astype(jnp.bfloat16)

        fwd = {}
        for d in range(1, Z):
            tz = lax.rem(mz + d, Z)
            sbuf[d - 1] = lax.dot(
                pp_ref[tz], xl, preferred_element_type=jnp.float32
            ).astype(jnp.bfloat16)
            r = pltpu.make_async_remote_copy(
                src_ref=sbuf.at[d - 1],
                dst_ref=abuf.at[d - 1],
                send_sem=fwd_send.at[d - 1],
                recv_sem=fwd_recv.at[d - 1],
                device_id=(mx, my, tz),
                device_id_type=pl.DeviceIdType.MESH,
            )
            r.start()
            fwd[d] = r

        w1_dma.wait()
        w2_dma.wait()
        w1s = [w1v[k].astype(jnp.bfloat16) for k in range(E)]
        w2s = [w2v[k].astype(jnp.bfloat16) for k in range(E)]

        def ffn(win):
            outs = []
            for k in range(E):
                xk = win[k * ECAP:(k + 1) * ECAP, :]
                h1 = lax.dot(xk, w1s[k], preferred_element_type=jnp.float32)
                hk = jnp.maximum(h1, 0.0).astype(jnp.bfloat16)
                outs.append(
                    lax.dot(hk, w2s[k], preferred_element_type=jnp.float32)
                )
            return jnp.concatenate(outs, axis=0).astype(jnp.bfloat16)

        own = ffn(
            lax.dot(
                pp_ref[mz], xl, preferred_element_type=jnp.float32
            ).astype(jnp.bfloat16)
        )

        ret = {}
        for j in range(1, Z):
            wr = pltpu.make_async_remote_copy(
                src_ref=sbuf.at[0], dst_ref=abuf.at[j - 1],
                send_sem=fwd_send.at[0], recv_sem=fwd_recv.at[j - 1],
                device_id=(mx, my, mz),
                device_id_type=pl.DeviceIdType.MESH,
            )
            wr.wait_recv()
            obuf[j - 1] = ffn(abuf[j - 1])
            sz = lax.rem(mz + Z - j, Z)
            r = pltpu.make_async_remote_copy(
                src_ref=obuf.at[j - 1],
                dst_ref=rbuf.at[j - 1],
                send_sem=ret_send.at[j - 1],
                recv_sem=ret_recv.at[j - 1],
                device_id=(mx, my, sz),
                device_id_type=pl.DeviceIdType.MESH,
            )
            r.start()
            ret[j] = r

        acc = lax.dot_general(
            pp_ref[mz], own,
            dimension_numbers=(((0,), (0,)), ((), ())),
            preferred_element_type=jnp.float32,
        )
        for d in range(1, Z):
            wr = pltpu.make_async_remote_copy(
                src_ref=obuf.at[0], dst_ref=rbuf.at[d - 1],
                send_sem=ret_send.at[0], recv_sem=ret_recv.at[d - 1],
                device_id=(mx, my, mz),
                device_id_type=pl.DeviceIdType.MESH,
            )
            wr.wait_recv()
            tz = lax.rem(mz + d, Z)
            acc = acc + lax.dot_general(
                pp_ref[tz], rbuf[d - 1],
                dimension_numbers=(((0,), (0,)), ((), ())),
                preferred_element_type=jnp.float32,
            )
        out_ref[...] = acc

        for d in range(1, Z):
            fwd[d].wait_send()
            ret[d].wait_send()

    return pl.pallas_call(
        body,
        out_shape=jax.ShapeDtypeStruct((T, D), jnp.float32),
        in_specs=[
            pl.BlockSpec(memory_space=pltpu.VMEM),
            pl.BlockSpec(memory_space=pltpu.MemorySpace.HBM),
            pl.BlockSpec(memory_space=pltpu.MemorySpace.HBM),
            pl.BlockSpec(memory_space=pltpu.VMEM),
        ],
        out_specs=pl.BlockSpec(memory_space=pltpu.VMEM),
        scratch_shapes=[
            pltpu.VMEM((Z - 1, PCAP, D), jnp.bfloat16),
            pltpu.VMEM((Z - 1, PCAP, D), jnp.bfloat16),
            pltpu.VMEM((Z - 1, PCAP, D), jnp.bfloat16),
            pltpu.VMEM((Z - 1, PCAP, D), jnp.bfloat16),
            pltpu.VMEM((E, D, F), jnp.float32),
            pltpu.VMEM((E, F, D), jnp.float32),
            pltpu.SemaphoreType.DMA((Z - 1,)),
            pltpu.SemaphoreType.DMA((Z - 1,)),
            pltpu.SemaphoreType.DMA((Z - 1,)),
            pltpu.SemaphoreType.DMA((Z - 1,)),
            pltpu.SemaphoreType.DMA((2,)),
        ],
        compiler_params=pltpu.CompilerParams(collective_id=0),
    )(x, W1, W2, ppair)
